# baseline (speedup 1.0000x reference)
import functools

import jax
import jax.numpy as jnp
import numpy as np
from jax import lax
from jax.experimental import pallas as pl
from jax.experimental.pallas import tpu as pltpu
from jax.experimental.pallas import tpu_sc as plsc

_CT = 0.1
_K = 32
_STUFF = 64
_LD = 1000
_NC = 12
_H = 256
_W = 256
_G = 64
_VS = 0.09375
_DMIN = 0.4
_DMAX = 6.0
_TRUNC = 3.0
_GXC = 8
_L = 16
_NW = 32
_RPW = _H // _NW

_NINF = np.float32(-np.inf)
_INF = np.float32(np.inf)


def _stage_a(heat_ref, sem_ref, pan_s_ref, cy_ref, cx_ref, val_ref,
             cls_ref, cyb_ref, cxb_ref, payb_ref, anyv_ref):
    heat = heat_ref[0, 0]
    ninf_row = jnp.full((1, _W), _NINF, jnp.float32)
    up = jnp.concatenate([heat[1:, :], ninf_row], axis=0)
    dn = jnp.concatenate([ninf_row, heat[:-1, :]], axis=0)
    vm = jnp.maximum(jnp.maximum(up, dn), heat)
    ninf_col = jnp.full((_H, 1), _NINF, jnp.float32)
    lf = jnp.concatenate([vm[:, 1:], ninf_col], axis=1)
    rt = jnp.concatenate([ninf_col, vm[:, :-1]], axis=1)
    pooled = jnp.maximum(jnp.maximum(lf, rt), vm)
    keep = (heat == pooled) & (heat > _CT)
    masked = jnp.where(keep, heat, _NINF)

    best = sem_ref[0, 0]
    bi = jnp.zeros((_H, _W), jnp.int32)
    for c in range(1, _NC):
        v = sem_ref[0, c]
        p = v > best
        best = jnp.where(p, v, best)
        bi = jnp.where(p, jnp.int32(c), bi)

    cm = jnp.zeros((_H, _W), jnp.int32)
    for c in range(_NC):
        cnt = jnp.sum((bi == c).astype(jnp.int32))
        cm = jnp.where(bi == c, cnt, cm)
    small = cm < _STUFF

    row_i = lax.broadcasted_iota(jnp.int32, (_H, _W), 0)
    col_i = lax.broadcasted_iota(jnp.int32, (_H, _W), 1)
    iota2d = row_i * _W + col_i
    big = jnp.int32(1 << 30)
    m_arr = masked
    cys, cxs, vals, clss = [], [], [], []
    for k in range(_K):
        m = jnp.max(m_arr)
        idx = jnp.min(jnp.where(m_arr == m, iota2d, big))
        hit = iota2d == idx
        cy = idx // _W
        cx = idx % _W
        v = m > _CT
        ck = jnp.max(jnp.where(hit, bi, 0))
        ck = jnp.where(v, ck, jnp.int32(0))
        m_arr = jnp.where(hit, _NINF, m_arr)
        cys.append(cy)
        cxs.append(cx)
        vals.append(v)
        clss.append(ck)

    is_thing = (bi >= 1) & (bi <= 8)
    pan_stuff = jnp.where(small, jnp.int32(0), bi * _LD)
    pan_s_ref[0] = jnp.where(is_thing, jnp.int32(-1), pan_stuff)

    cyb_rows, cxb_rows, pay_rows = [], [], []
    for k in range(_K):
        cy_ref[0, 0, k] = cys[k]
        cx_ref[0, 0, k] = cxs[k]
        val_ref[0, 0, k] = vals[k].astype(jnp.int32)
        cls_ref[0, 0, k] = clss[k]
        cyf = jnp.where(vals[k], cys[k].astype(jnp.float32), _INF)
        cxf = jnp.where(vals[k], cxs[k].astype(jnp.float32), _INF)
        cyb_rows.append(jnp.full((1, _L), cyf, jnp.float32))
        cxb_rows.append(jnp.full((1, _L), cxf, jnp.float32))
        pay_rows.append(jnp.full((1, _L), clss[k] * _LD + jnp.int32(k + 1),
                                 jnp.int32))
    cyb_ref[0] = jnp.concatenate(cyb_rows, axis=0)
    cxb_ref[0] = jnp.concatenate(cxb_rows, axis=0)
    payb_ref[0] = jnp.concatenate(pay_rows, axis=0)
    any_valid = vals[0]
    for k in range(1, _K):
        any_valid = any_valid | vals[k]
    anyv_ref[0] = jnp.full((1, _L), any_valid.astype(jnp.int32), jnp.int32)


def _sc_assign(nb, off_hbm, base_hbm, cyb_hbm, cxb_hbm, payb_hbm,
               anyv_hbm, pan2_hbm, offy_v, offx_v, base_v, out_v, cy_v,
               cx_v, pay_v, any_v):
    wid = lax.axis_index("s") * 2 + lax.axis_index("c")
    r0 = wid * _RPW
    lane = lax.iota(jnp.int32, _L)
    for b in range(nb):
        pltpu.sync_copy(off_hbm.at[b, 0, pl.ds(r0, _RPW), :], offy_v)
        pltpu.sync_copy(off_hbm.at[b, 1, pl.ds(r0, _RPW), :], offx_v)
        pltpu.sync_copy(base_hbm.at[b, pl.ds(r0, _RPW), :], base_v)
        pltpu.sync_copy(cyb_hbm.at[b], cy_v)
        pltpu.sync_copy(cxb_hbm.at[b], cx_v)
        pltpu.sync_copy(payb_hbm.at[b], pay_v)
        pltpu.sync_copy(anyv_hbm.at[b, 0], any_v)
        cyk = [cy_v[k] for k in range(_K)]
        cxk = [cx_v[k] for k in range(_K)]
        payk = [pay_v[k] for k in range(_K)]
        av = any_v[...] != 0

        def body(i, carry):
            r = i // _L
            c = (i % _L) * _L
            offy = offy_v[r, pl.ds(c, _L)]
            offx = offx_v[r, pl.ds(c, _L)]
            ly = (r0 + r).astype(jnp.float32) + offy
            lx = (c + lane).astype(jnp.float32) + offx
            dy = ly - cyk[0]
            dx = lx - cxk[0]
            best_d = dy * dy + dx * dx
            best_pay = payk[0]
            for k in range(1, _K):
                dy = ly - cyk[k]
                dx = lx - cxk[k]
                dk = dy * dy + dx * dx
                p = dk < best_d
                best_d = jnp.where(p, dk, best_d)
                best_pay = jnp.where(p, payk[k], best_pay)
            basev = base_v[r, pl.ds(c, _L)]
            thing = jnp.where(av, best_pay, jnp.int32(0))
            out_v[r, pl.ds(c, _L)] = jnp.where(basev < 0, thing, basev)
            return carry

        lax.fori_loop(0, _RPW * _W // _L, body, jnp.int32(0))
        pltpu.sync_copy(out_v, pan2_hbm.at[b, pl.ds(r0, _RPW), :])


def _stage_b(geo_ref, occ_ref, sem3_ref, off3_ref, intr_ref, cy_ref,
             cx_ref, val_ref, cls_ref, pan3_ref, geo_out_ref):
    j = pl.program_id(1)
    geo = geo_ref[0, 0]
    occ = occ_ref[0, 0]
    geo = jnp.where(occ <= 0.0, jnp.float32(_TRUNC), geo)
    geo_out_ref[0, 0] = geo
    surface = jnp.abs(geo) < 1.5

    best = sem3_ref[0, 0]
    s3 = jnp.zeros((_GXC, _G, _G), jnp.int32)
    for c in range(1, _NC):
        v = sem3_ref[0, c]
        p = v > best
        best = jnp.where(p, v, best)
        s3 = jnp.where(p, jnp.int32(c), s3)

    gx = lax.broadcasted_iota(jnp.int32, (_GXC, _G, _G), 0).astype(
        jnp.float32) + (j * _GXC).astype(jnp.float32)
    gy = lax.broadcasted_iota(jnp.int32, (_GXC, _G, _G), 1).astype(
        jnp.float32)
    gz = lax.broadcasted_iota(jnp.int32, (_GXC, _G, _G), 2).astype(
        jnp.float32)
    vx = (gx + off3_ref[0, 0] - _G / 2.0) * _VS
    vy = (gy + off3_ref[0, 1] - _G / 2.0) * _VS
    vz = jnp.clip(_DMIN + (gz + off3_ref[0, 2]) * _VS, _DMIN, _DMAX)
    fx = intr_ref[0, 0, 0]
    fy = intr_ref[0, 0, 1]
    cxi = intr_ref[0, 0, 2]
    cyi = intr_ref[0, 0, 3]
    u = fx * vx / vz + cxi
    v = fy * vy / vz + cyi

    best_d = None
    best_pay = None
    any_valid = None
    for k in range(_K):
        vk = val_ref[0, 0, k] != 0
        cyf = jnp.where(vk, cy_ref[0, 0, k].astype(jnp.float32), _INF)
        cxf = jnp.where(vk, cx_ref[0, 0, k].astype(jnp.float32), _INF)
        dk = (v - cyf) ** 2 + (u - cxf) ** 2
        pay = cls_ref[0, 0, k] * _LD + jnp.int32(k + 1)
        if best_d is None:
            best_d = dk
            best_pay = jnp.broadcast_to(pay, (_GXC, _G, _G))
            any_valid = vk
        else:
            p = dk < best_d
            best_d = jnp.where(p, dk, best_d)
            best_pay = jnp.where(p, pay, best_pay)
            any_valid = any_valid | vk

    is_thing3 = (s3 >= 1) & (s3 <= 8)
    pan_thing = jnp.where(any_valid, best_pay, jnp.int32(0))
    pan = jnp.where(is_thing3, pan_thing, s3 * _LD)
    pan3_ref[0] = jnp.where(surface, pan, jnp.int32(0))


def kernel(semantic2d, center2d, offset2d, geometry, occupancy3d,
           semantic3d, offset3d, intrinsic):
    B = semantic2d.shape[0]

    smem_row = pl.BlockSpec((1, 1, _K), lambda b: (b, 0, 0),
                            memory_space=pltpu.SMEM)
    base, cy, cx, val, cls, cyb, cxb, payb, anyv = pl.pallas_call(
        _stage_a,
        grid=(B,),
        in_specs=[
            pl.BlockSpec((1, 1, _H, _W), lambda b: (b, 0, 0, 0)),
            pl.BlockSpec((1, _NC, _H, _W), lambda b: (b, 0, 0, 0)),
        ],
        out_specs=[
            pl.BlockSpec((1, _H, _W), lambda b: (b, 0, 0)),
            smem_row, smem_row, smem_row, smem_row,
            pl.BlockSpec((1, _K, _L), lambda b: (b, 0, 0)),
            pl.BlockSpec((1, _K, _L), lambda b: (b, 0, 0)),
            pl.BlockSpec((1, _K, _L), lambda b: (b, 0, 0)),
            pl.BlockSpec((1, 1, _L), lambda b: (b, 0, 0)),
        ],
        out_shape=[
            jax.ShapeDtypeStruct((B, _H, _W), jnp.int32),
            jax.ShapeDtypeStruct((B, 1, _K), jnp.int32),
            jax.ShapeDtypeStruct((B, 1, _K), jnp.int32),
            jax.ShapeDtypeStruct((B, 1, _K), jnp.int32),
            jax.ShapeDtypeStruct((B, 1, _K), jnp.int32),
            jax.ShapeDtypeStruct((B, _K, _L), jnp.float32),
            jax.ShapeDtypeStruct((B, _K, _L), jnp.float32),
            jax.ShapeDtypeStruct((B, _K, _L), jnp.int32),
            jax.ShapeDtypeStruct((B, 1, _L), jnp.int32),
        ],
        compiler_params=pltpu.CompilerParams(
            dimension_semantics=("arbitrary",)),
    )(center2d, semantic2d)

    mesh = plsc.VectorSubcoreMesh(core_axis_name="c", subcore_axis_name="s")
    pan2 = pl.kernel(
        functools.partial(_sc_assign, B),
        out_type=jax.ShapeDtypeStruct((B, _H, _W), jnp.int32),
        mesh=mesh,
        scratch_types=[
            pltpu.VMEM((_RPW, _W), jnp.float32),
            pltpu.VMEM((_RPW, _W), jnp.float32),
            pltpu.VMEM((_RPW, _W), jnp.int32),
            pltpu.VMEM((_RPW, _W), jnp.int32),
            pltpu.VMEM((_K, _L), jnp.float32),
            pltpu.VMEM((_K, _L), jnp.float32),
            pltpu.VMEM((_K, _L), jnp.int32),
            pltpu.VMEM((_L,), jnp.int32),
        ],
    )(offset2d, base, cyb, cxb, payb, anyv)

    intr = jnp.stack([intrinsic[:, 0, 0], intrinsic[:, 1, 1],
                      intrinsic[:, 0, 2], intrinsic[:, 1, 2]],
                     axis=1).reshape(B, 1, 4)

    ngx = _G // _GXC
    pan3, geo_out = pl.pallas_call(
        _stage_b,
        grid=(B, ngx),
        in_specs=[
            pl.BlockSpec((1, 1, _GXC, _G, _G), lambda b, g: (b, 0, g, 0, 0)),
            pl.BlockSpec((1, 1, _GXC, _G, _G), lambda b, g: (b, 0, g, 0, 0)),
            pl.BlockSpec((1, _NC, _GXC, _G, _G),
                         lambda b, g: (b, 0, g, 0, 0)),
            pl.BlockSpec((1, 3, _GXC, _G, _G), lambda b, g: (b, 0, g, 0, 0)),
            pl.BlockSpec((1, 1, 4), lambda b, g: (b, 0, 0),
                         memory_space=pltpu.SMEM),
            pl.BlockSpec((1, 1, _K), lambda b, g: (b, 0, 0),
                         memory_space=pltpu.SMEM),
            pl.BlockSpec((1, 1, _K), lambda b, g: (b, 0, 0),
                         memory_space=pltpu.SMEM),
            pl.BlockSpec((1, 1, _K), lambda b, g: (b, 0, 0),
                         memory_space=pltpu.SMEM),
            pl.BlockSpec((1, 1, _K), lambda b, g: (b, 0, 0),
                         memory_space=pltpu.SMEM),
        ],
        out_specs=[
            pl.BlockSpec((1, _GXC, _G, _G), lambda b, g: (b, g, 0, 0)),
            pl.BlockSpec((1, 1, _GXC, _G, _G), lambda b, g: (b, 0, g, 0, 0)),
        ],
        out_shape=[
            jax.ShapeDtypeStruct((B, _G, _G, _G), jnp.int32),
            jax.ShapeDtypeStruct((B, 1, _G, _G, _G), jnp.float32),
        ],
        compiler_params=pltpu.CompilerParams(
            dimension_semantics=("arbitrary", "arbitrary")),
    )(geometry, occupancy3d, semantic3d, offset3d, intr, cy, cx, val, cls)

    cyv = cy[:, 0, :]
    cxv = cx[:, 0, :]
    valid = val[:, 0, :] != 0
    centers = jnp.stack([cyv, cxv], axis=-1)
    cp = jnp.where(valid[..., None], centers, -1)
    return pan2, pan3, cp, cls[:, 0, :], geo_out
